# Initial kernel scaffold; baseline (speedup 1.0000x reference)
import jax
import jax.numpy as jnp
from jax.experimental import pallas as pl


def kernel(x):
    raise NotImplementedError("write your pallas kernel here")


# speedup vs baseline: 2.7111x; 2.7111x over previous
import functools

import jax
import jax.numpy as jnp
import numpy as np
from jax import lax
from jax.experimental import pallas as pl
from jax.experimental.pallas import tpu as pltpu
from jax.experimental.pallas import tpu_sc as plsc

_ANCHORS = np.array([[10.0, 13.0], [16.0, 30.0], [33.0, 23.0]], dtype=np.float32)
_N = 16
_A = 3
_H = 64
_W = 64
_NCLS = 80
_NB = _A * _H * _W
_K = 300
_PAD = 512
_MAXDET = 100
_CONF = 0.5
_NMS_T = 0.4
_IMG = 512.0
_ONE_BITS = np.int32(np.float32(1.0).view(np.int32))


def _decode_body(x_ref, s_ref, x1_ref, y1_ref, x2_ref, y2_ref, c_ref, e_ref):
    xb = x_ref[0]
    gx = lax.broadcasted_iota(jnp.int32, (_H, _W), 1).astype(jnp.float32)
    gy = lax.broadcasted_iota(jnp.int32, (_H, _W), 0).astype(jnp.float32)
    scores = []
    for a in range(_A):
        base = a * (_NCLS + 5)
        tx = xb[base + 0]
        ty = xb[base + 1]
        tw = xb[base + 2]
        th = xb[base + 3]
        tobj = xb[base + 4]
        cls_blk = xb[base + 5:base + 5 + _NCLS]
        cmax = jnp.max(cls_blk, axis=0)
        cid = jnp.argmax(cls_blk, axis=0).astype(jnp.float32)
        px = jax.nn.sigmoid(tx) + gx
        py = jax.nn.sigmoid(ty) + gy
        pw = jnp.exp(tw) * _ANCHORS[a, 0]
        ph = jnp.exp(th) * _ANCHORS[a, 1]
        bx = (px * _IMG) * _IMG
        by = (py * _IMG) * _IMG
        bw = (pw * _IMG) * _IMG
        bh = (ph * _IMG) * _IMG
        score = jax.nn.sigmoid(tobj) * jax.nn.sigmoid(cmax)
        s_ref[0, a] = score
        x1_ref[0, a] = bx - bw / 2.0
        y1_ref[0, a] = by - bh / 2.0
        x2_ref[0, a] = bx + bw / 2.0
        y2_ref[0, a] = by + bh / 2.0
        c_ref[0, a] = cid
        scores.append(score)

    s3 = jnp.stack(scores, axis=0)
    bits = lax.bitcast_convert_type(s3, jnp.int32)

    def b1(_, carry):
        lo, hi = carry
        mid = (lo + hi) // 2
        cnt = jnp.sum((bits >= mid).astype(jnp.int32))
        ok = cnt >= _K
        return jnp.where(ok, mid, lo), jnp.where(ok, hi, mid)

    lo, _ = lax.fori_loop(0, 31, b1, (jnp.int32(0), _ONE_BITS + jnp.int32(1)))
    tbits = lo
    cnt_gt = jnp.sum((bits >= tbits + 1).astype(jnp.int32))
    need = _K - cnt_gt

    ia = lax.broadcasted_iota(jnp.int32, (_A, _H, _W), 0)
    iy = lax.broadcasted_iota(jnp.int32, (_A, _H, _W), 1)
    ix = lax.broadcasted_iota(jnp.int32, (_A, _H, _W), 2)
    lin = ia * (_H * _W) + iy * _W + ix
    at_t = bits == tbits

    def b2(_, carry):
        lo2, hi2 = carry
        mid = (lo2 + hi2) // 2
        cnt = jnp.sum((at_t & (lin <= mid)).astype(jnp.int32))
        ok = cnt >= need
        return jnp.where(ok, lo2, mid), jnp.where(ok, mid, hi2)

    _, cstar = lax.fori_loop(0, 15, b2, (jnp.int32(-1), jnp.int32(_NB - 1)))
    elig = (bits > tbits) | (at_t & (lin <= cstar))

    e_i = elig.astype(jnp.int32)
    c = e_i
    for k in (1, 2, 4, 8, 16, 32):
        c = c + jnp.concatenate(
            [jnp.zeros((_A, _H, k), jnp.int32), c[:, :, :-k]], axis=2)
    rowtot = c[:, :, _W - 1:_W]
    r = rowtot
    for k in (1, 2, 4, 8, 16, 32):
        r = r + jnp.concatenate(
            [jnp.zeros((_A, k, 1), jnp.int32), r[:, :-k, :]], axis=1)
    atot = r[:, _H - 1:_H, :]
    aoff = jnp.concatenate(
        [jnp.zeros((1, 1, 1), jnp.int32), atot[0:1], atot[0:1] + atot[1:2]],
        axis=0)
    pexcl = (c - e_i) + (r - rowtot) + aoff
    e_ref[0] = jnp.where(elig, pexcl, jnp.int32(_PAD - 1))


def _decode(x):
    f = jax.ShapeDtypeStruct((_N, _A, _H, _W), jnp.float32)
    fi = jax.ShapeDtypeStruct((_N, _A, _H, _W), jnp.int32)
    fld = pl.BlockSpec((1, _A, _H, _W), lambda n: (n, 0, 0, 0))
    return pl.pallas_call(
        _decode_body,
        grid=(_N,),
        in_specs=[pl.BlockSpec((1, _A * (_NCLS + 5), _H, _W), lambda n: (n, 0, 0, 0))],
        out_specs=[fld] * 7,
        out_shape=[f] * 6 + [fi],
    )(x)


def _compact_body(s_h, x1_h, y1_h, x2_h, y2_h, c_h, d_h,
                  os_h, ox1_h, oy1_h, ox2_h, oy2_h, oc_h,
                  s_v, x1_v, y1_v, x2_v, y2_v, c_v, d_v,
                  bs, bx1, by1, bx2, by2, bc):
    wid = lax.axis_index("s") * 2 + lax.axis_index("c")

    @pl.when(wid < _N)
    def _():
        img = wid
        pltpu.sync_copy(s_h.at[img], s_v)
        pltpu.sync_copy(x1_h.at[img], x1_v)
        pltpu.sync_copy(y1_h.at[img], y1_v)
        pltpu.sync_copy(x2_h.at[img], x2_v)
        pltpu.sync_copy(y2_h.at[img], y2_v)
        pltpu.sync_copy(c_h.at[img], c_v)
        pltpu.sync_copy(d_h.at[img], d_v)

        zeros = jnp.zeros((16,), jnp.float32)

        def zbody(j, carry):
            sl = pl.ds(j * 16, 16)
            bs[sl] = zeros
            bx1[sl] = zeros
            by1[sl] = zeros
            bx2[sl] = zeros
            by2[sl] = zeros
            bc[sl] = zeros
            return carry

        lax.fori_loop(0, _PAD // 16, zbody, 0)

        def body(i, carry):
            sl = pl.ds(i * 16, 16)
            pos = d_v[sl]
            plsc.store_scatter(bs, [pos], s_v[sl])
            plsc.store_scatter(bx1, [pos], x1_v[sl])
            plsc.store_scatter(by1, [pos], y1_v[sl])
            plsc.store_scatter(bx2, [pos], x2_v[sl])
            plsc.store_scatter(by2, [pos], y2_v[sl])
            plsc.store_scatter(bc, [pos], c_v[sl])
            return carry

        lax.fori_loop(0, _NB // 16, body, 0)

        tail = pl.ds(_PAD - 16, 16)
        bs[tail] = zeros
        bx1[tail] = zeros
        by1[tail] = zeros
        bx2[tail] = zeros
        by2[tail] = zeros
        bc[tail] = zeros

        pltpu.sync_copy(bs, os_h.at[img])
        pltpu.sync_copy(bx1, ox1_h.at[img])
        pltpu.sync_copy(by1, oy1_h.at[img])
        pltpu.sync_copy(bx2, ox2_h.at[img])
        pltpu.sync_copy(by2, oy2_h.at[img])
        pltpu.sync_copy(bc, oc_h.at[img])


def _compact(s, x1, y1, x2, y2, c, d):
    out = jax.ShapeDtypeStruct((_N, _PAD), jnp.float32)
    mesh = plsc.VectorSubcoreMesh(core_axis_name="c", subcore_axis_name="s")
    big = pltpu.VMEM((_NB,), jnp.float32)
    bigi = pltpu.VMEM((_NB,), jnp.int32)
    small = pltpu.VMEM((_PAD,), jnp.float32)
    return pl.kernel(
        _compact_body,
        out_type=[out] * 6,
        mesh=mesh,
        scratch_types=[big] * 6 + [bigi] + [small] * 6,
        compiler_params=pltpu.CompilerParams(needs_layout_passes=False),
    )(s, x1, y1, x2, y2, c, d)


def _nms_body(s_ref, x1_ref, y1_ref, x2_ref, y2_ref, c_ref,
              ox1, oy1, ox2, oy2, os_, oc):
    s = s_ref[...]
    x1 = x1_ref[...]
    y1 = y1_ref[...]
    x2 = x2_ref[...]
    y2 = y2_ref[...]
    cid = c_ref[...]
    area = (x2 - x1) * (y2 - y1)
    iota = lax.broadcasted_iota(jnp.int32, (_N, _PAD), 1)
    suppr = s < _CONF

    def ext(onehot, arr):
        return jnp.sum(jnp.where(onehot, arr, 0.0), axis=1, keepdims=True)

    for j in range(_MAXDET):
        masked = jnp.where(suppr, -1.0, s)
        m = jnp.max(masked, axis=1, keepdims=True)
        i_min = jnp.min(jnp.where(masked == m, iota, _PAD), axis=1, keepdims=True)
        onehot = iota == i_min
        valid = m >= 0.0
        bx1 = ext(onehot, x1)
        by1 = ext(onehot, y1)
        bx2 = ext(onehot, x2)
        by2 = ext(onehot, y2)
        bc = ext(onehot, cid)
        barea = ext(onehot, area)
        xx1 = jnp.maximum(bx1, x1)
        yy1 = jnp.maximum(by1, y1)
        xx2 = jnp.minimum(bx2, x2)
        yy2 = jnp.minimum(by2, y2)
        inter = jnp.maximum(xx2 - xx1, 0.0) * jnp.maximum(yy2 - yy1, 0.0)
        iou = inter / (barea + area - inter + 1e-16)
        suppr = suppr | (iou > _NMS_T) | onehot
        col = slice(j, j + 1)
        ox1[:, col] = jnp.where(valid, bx1, 0.0)
        oy1[:, col] = jnp.where(valid, by1, 0.0)
        ox2[:, col] = jnp.where(valid, bx2, 0.0)
        oy2[:, col] = jnp.where(valid, by2, 0.0)
        os_[:, col] = jnp.where(valid, m, 0.0)
        oc[:, col] = jnp.where(valid, bc, 0.0)


def _nms(s, x1, y1, x2, y2, c):
    out = jax.ShapeDtypeStruct((_N, 128), jnp.float32)
    return pl.pallas_call(
        _nms_body,
        out_shape=[out] * 6,
    )(s, x1, y1, x2, y2, c)


def kernel(x):
    s, x1, y1, x2, y2, c, d = _decode(x)
    rs = lambda a: a.reshape(_N, _NB)
    cs, cx1, cy1, cx2, cy2, cc = _compact(rs(s), rs(x1), rs(y1), rs(x2),
                                          rs(y2), rs(c), rs(d))
    ox1, oy1, ox2, oy2, osc, ocl = _nms(cs, cx1, cy1, cx2, cy2, cc)
    rows = jnp.stack([ox1[:, :_MAXDET], oy1[:, :_MAXDET], ox2[:, :_MAXDET],
                      oy2[:, :_MAXDET], osc[:, :_MAXDET], ocl[:, :_MAXDET]],
                     axis=-1)
    return rows
